# Initial kernel scaffold; baseline (speedup 1.0000x reference)
import jax
import jax.numpy as jnp
from jax.experimental import pallas as pl


def kernel(y, y_true):
    raise NotImplementedError("write your pallas kernel here")


# speedup vs baseline: 2.2533x; 2.2533x over previous
import functools

import jax
import jax.numpy as jnp
from jax import lax
from jax.experimental import pallas as pl
from jax.experimental.pallas import tpu as pltpu
from jax.experimental.pallas import tpu_sc as plsc

C = 16
NC = 2
NS = 16
NW = NC * NS
MOMENTUM = 0.9
ALPHA = 0.5
LN2 = 0.6931471805599453

CHUNK = 2048
GROUP = 16


def _log_f32(s):
    bits = lax.bitcast_convert_type(s, jnp.int32)
    e = lax.shift_right_logical(bits, 23) - 127
    mant_bits = lax.bitwise_or(lax.bitwise_and(bits, 0x007FFFFF), 0x3F800000)
    mf = lax.bitcast_convert_type(mant_bits, jnp.float32)
    z = (mf - 1.0) / (mf + 1.0)
    z2 = z * z
    poly = 1.0 + z2 * (1.0 / 3.0 + z2 * (1.0 / 5.0 + z2 * (1.0 / 7.0)))
    return e.astype(jnp.float32) * LN2 + 2.0 * z * poly


def _sc_partials(y_flat, y_true):
    batch = y_true.shape[0]
    rows_per_w = batch // NW
    n_chunks = rows_per_w // CHUNK
    n_groups = CHUNK // GROUP

    mesh = plsc.VectorSubcoreMesh(core_axis_name="c", subcore_axis_name="s")

    @functools.partial(
        pl.kernel,
        out_type=jax.ShapeDtypeStruct((NW, 4 * C), jnp.float32),
        mesh=mesh,
        compiler_params=pltpu.CompilerParams(needs_layout_passes=False),
        scratch_types=[
            pltpu.VMEM((CHUNK * C,), jnp.float32),
            pltpu.VMEM((CHUNK,), jnp.int32),
            pltpu.VMEM((4 * C,), jnp.float32),
        ],
    )
    def sc_kernel(y_hbm, t_hbm, out_hbm, yv, tv, accv):
        cid = lax.axis_index("c")
        sid = lax.axis_index("s")
        wid = sid * NC + cid
        base = wid * rows_per_w

        zeros = jnp.zeros((C,), jnp.float32)
        ones = jnp.ones((C,), jnp.float32)
        rowoff = lax.iota(jnp.int32, C) * C
        for k in range(4):
            accv[pl.ds(k * C, C)] = zeros

        def chunk_body(ci, _):
            r0 = base + ci * CHUNK
            pltpu.sync_copy(y_hbm.at[pl.ds(r0 * C, CHUNK * C)], yv)
            pltpu.sync_copy(t_hbm.at[pl.ds(r0, CHUNK)], tv)

            def group_body(g, carry):
                gbase = g * (GROUP * C)
                idx0 = rowoff + gbase
                t = tv[pl.ds(g * GROUP, GROUP)]
                us = [plsc.load_gather(yv, [idx0 + c]) for c in range(C)]
                m = us[0]
                for c in range(1, C):
                    m = jnp.maximum(m, us[c])
                pred = jnp.full((C,), C - 1, jnp.int32)
                for c in range(C - 2, -1, -1):
                    pred = jnp.where(us[c] == m, jnp.int32(c), pred)
                s = zeros
                for c in range(C):
                    s = s + jnp.exp(us[c] - m)
                logp = plsc.load_gather(yv, [idx0 + t]) - m - _log_f32(s)
                plsc.addupdate_scatter(accv, [t], ones)
                plsc.addupdate_scatter(accv, [pred + C], ones)
                plsc.addupdate_scatter(accv, [pred + 2 * C], ones,
                                       mask=pred == t)
                plsc.addupdate_scatter(accv, [t + 3 * C], logp)
                return carry

            lax.fori_loop(0, n_groups, group_body, 0, unroll=False)
            return _

        lax.fori_loop(0, n_chunks, chunk_body, 0, unroll=False)
        pltpu.sync_copy(accv, out_hbm.at[wid])

    return sc_kernel(y_flat, y_true)


def _epilogue_kernel(parts_ref, o_ref):
    parts = parts_ref[...]
    acc = jnp.sum(parts, axis=0, keepdims=True)
    rec = acc[:, 0:C]
    prc = acc[:, C:2 * C]
    rgt = acc[:, 2 * C:3 * C]
    ssum = acc[:, 3 * C:4 * C]
    p = rgt / prc
    r = rgt / rec
    focal = 1.0 - p * r / (ALPHA * p + (1.0 - ALPHA) * r)
    w = (1.0 - MOMENTUM) * focal
    num = jnp.sum(w * ssum)
    den = jnp.sum(w * rec)
    o_ref[0, 0] = -num / den


def kernel(y, y_true):
    parts = _sc_partials(y.reshape(-1), y_true)
    loss = pl.pallas_call(
        _epilogue_kernel,
        out_shape=jax.ShapeDtypeStruct((1, 1), jnp.float32),
        out_specs=pl.BlockSpec(memory_space=pltpu.SMEM),
    )(parts)
    return loss[0, 0]
